# baseline (speedup 1.0000x reference)
import jax
from jax import lax
import jax.numpy as jnp
from jax.experimental import pallas as pl
from jax.experimental.pallas import tpu as pltpu
from jax.experimental.pallas import tpu_sc as plsc

_N_NODES = 100000
_DIM = 128
_NUM_SPECIES = 128
_RB = 12544
_TC_BLOCKS = 7
_TC_ROWS = _RB * _TC_BLOCKS
_SC_ROWS = _N_NODES - _TC_ROWS
_SC_WINDOW = 128
_SC_PAD = 12288


def _sc_gather_tail(embeddings, idx2d):
    mesh = plsc.VectorSubcoreMesh(
        core_axis_name="core", subcore_axis_name="subcore"
    )

    @pl.kernel(
        out_type=jax.ShapeDtypeStruct((_SC_ROWS, _DIM), embeddings.dtype),
        mesh=mesh,
    )
    def gather_kernel(x_hbm, i_hbm, o_hbm):
        def body(i_vmem, o_vmem):
            pltpu.sync_copy(x_hbm.at[i_vmem.at[0]], o_vmem)

        pltpu.emit_pipeline(
            body,
            grid=(_SC_PAD // _SC_WINDOW,),
            in_specs=[
                pl.BlockSpec((1, _SC_WINDOW), index_map=lambda i: (0, i))
            ],
            out_specs=[
                pl.BlockSpec((_SC_WINDOW, _DIM), index_map=lambda i: (i, 0))
            ],
            core_axis_name=("core", "subcore"),
            dimension_semantics=(pltpu.PARALLEL,),
        )(i_hbm, o_hbm)

    return gather_kernel(embeddings, idx2d)


def _tc_lookup_head(idx_head, embeddings):
    idxp = idx_head.reshape(_TC_BLOCKS, 1, _RB)

    def body(i_ref, w_ref, o_ref):
        ids = i_ref[0, 0, :]
        onehot = (
            ids[:, None]
            == lax.broadcasted_iota(jnp.int32, (_RB, _NUM_SPECIES), 1)
        ).astype(jnp.bfloat16)
        w = w_ref[...]
        w_hi = w.astype(jnp.bfloat16)
        r1 = w - w_hi.astype(jnp.float32)
        w_mid = r1.astype(jnp.bfloat16)
        acc = jnp.dot(onehot, w_hi, preferred_element_type=jnp.float32)
        acc = acc + jnp.dot(onehot, w_mid, preferred_element_type=jnp.float32)
        o_ref[...] = acc

    return pl.pallas_call(
        body,
        grid=(_TC_BLOCKS,),
        in_specs=[
            pl.BlockSpec((1, 1, _RB), lambda i: (i, 0, 0)),
            pl.BlockSpec((_NUM_SPECIES, _DIM), lambda i: (0, 0)),
        ],
        out_specs=pl.BlockSpec((_RB, _DIM), lambda i: (i, 0)),
        out_shape=jax.ShapeDtypeStruct((_N_NODES, _DIM), jnp.float32),
    )(idxp, embeddings)


def kernel(node_specie, embeddings):
    idx_tail = jnp.pad(node_specie[_TC_ROWS:], (0, _SC_PAD - _SC_ROWS))
    sc_part = _sc_gather_tail(embeddings, idx_tail.reshape(1, _SC_PAD))
    tc_full = _tc_lookup_head(node_specie[:_TC_ROWS], embeddings)
    return lax.dynamic_update_slice(tc_full, sc_part, (_TC_ROWS, 0))
